# Initial kernel scaffold; baseline (speedup 1.0000x reference)
import jax
import jax.numpy as jnp
from jax.experimental import pallas as pl


def kernel(x, edge_index, edge_attr, W1, b1, W2, b2):
    raise NotImplementedError("write your pallas kernel here")


# speedup vs baseline: 10.3159x; 10.3159x over previous
import functools

import jax
import jax.numpy as jnp
from jax import lax
from jax.experimental import pallas as pl
from jax.experimental.pallas import tpu as pltpu
from jax.experimental.pallas import tpu_sc as plsc

L = 16
D = 128
K = 128
NC = 2
NS = 16
NW = NC * NS


def _agg_kernel(NP, NCH):
    ZR = NP // NS
    mesh = plsc.VectorSubcoreMesh(core_axis_name="c", subcore_axis_name="s")

    @functools.partial(
        pl.kernel,
        out_type=jax.ShapeDtypeStruct((NC, NP, D), jnp.float32),
        mesh=mesh,
        compiler_params=pltpu.CompilerParams(needs_layout_passes=False, use_tc_tiling_on_sc=False),
        scratch_types=[
            pltpu.VMEM((NP,), jnp.float32),
            pltpu.VMEM((K,), jnp.int32),
            pltpu.VMEM((K,), jnp.int32),
            pltpu.VMEM((K,), jnp.float32),
            pltpu.VMEM((K, D), jnp.float32),
            pltpu.VMEM_SHARED((NP, D), jnp.float32),
            pltpu.SemaphoreType.DMA,
        ],
    )
    def k(xw_hbm, dis_hbm, src_hbm, dst_hbm, ew_hbm, out_hbm,
          dis_v, src_v, dst_v, ew_v, rows, acc, sem):
        cid = lax.axis_index("c")
        sid = lax.axis_index("s")
        wid = cid * NS + sid
        zero = jnp.zeros((L,), jnp.float32)

        def zrow(i, _):
            for j in range(D // L):
                rows[i, pl.ds(j * L, L)] = zero
            return 0
        lax.fori_loop(0, K, zrow, 0)
        for c in range(ZR // K):
            pltpu.sync_copy(rows, acc.at[pl.ds(sid * ZR + c * K, K)])

        pltpu.sync_copy(dis_hbm.at[0], dis_v)
        plsc.subcore_barrier()

        def chunk(i, _):
            base = (wid * NCH + i) * K
            pltpu.sync_copy(src_hbm.at[pl.ds(base, K)], src_v)
            pltpu.sync_copy(dst_hbm.at[pl.ds(base, K)], dst_v)
            pltpu.sync_copy(ew_hbm.at[pl.ds(base, K)], ew_v)
            pltpu.async_copy(xw_hbm.at[src_v], rows, sem).wait()

            def grp(g, _):
                sl = pl.ds(g * L, L)
                s = src_v[sl]
                d = dst_v[sl]
                w = ew_v[sl]
                nvec = plsc.load_gather(dis_v, [s]) * w * plsc.load_gather(dis_v, [d])
                for e in range(L):
                    sc = nvec[e]
                    row = g * L + e
                    for j in range(D // L):
                        rj = pl.ds(j * L, L)
                        rows[row, rj] = rows[row, rj] * sc
                return 0
            lax.fori_loop(0, K // L, grp, 0)

            pltpu.sync_copy(rows, acc.at[dst_v], add=True)
            return 0
        lax.fori_loop(0, NCH, chunk, 0)

        plsc.subcore_barrier()
        pltpu.sync_copy(acc.at[pl.ds(sid * ZR, ZR)],
                        out_hbm.at[cid, pl.ds(sid * ZR, ZR)])

    return k


def _deg_kernel(NP, NCH):
    ZR = NP // NS
    mesh = plsc.VectorSubcoreMesh(core_axis_name="c", subcore_axis_name="s")

    @functools.partial(
        pl.kernel,
        out_type=jax.ShapeDtypeStruct((NC, NP, L), jnp.float32),
        mesh=mesh,
        compiler_params=pltpu.CompilerParams(needs_layout_passes=False, use_tc_tiling_on_sc=False),
        scratch_types=[
            pltpu.VMEM((K,), jnp.int32),
            pltpu.VMEM((K,), jnp.float32),
            pltpu.VMEM((K, L), jnp.float32),
            pltpu.VMEM((ZR, L), jnp.float32),
            pltpu.VMEM_SHARED((NP, L), jnp.float32),
        ],
    )
    def k(dst_hbm, ew_hbm, degp_hbm, dst_v, ew_v, msg, zbuf, deg_sp):
        cid = lax.axis_index("c")
        sid = lax.axis_index("s")
        wid = cid * NS + sid
        zero = jnp.zeros((L,), jnp.float32)

        def zz(i, _):
            zbuf[i] = zero
            return 0
        lax.fori_loop(0, ZR, zz, 0)
        pltpu.sync_copy(zbuf, deg_sp.at[pl.ds(sid * ZR, ZR)])
        plsc.subcore_barrier()

        def chunk(i, _):
            base = (wid * NCH + i) * K
            pltpu.sync_copy(dst_hbm.at[pl.ds(base, K)], dst_v)
            pltpu.sync_copy(ew_hbm.at[pl.ds(base, K)], ew_v)

            def grp(g, _):
                w = ew_v[pl.ds(g * L, L)]
                for e in range(L):
                    msg[g * L + e] = jnp.broadcast_to(w[e], (L,))
                return 0
            lax.fori_loop(0, K // L, grp, 0)

            pltpu.sync_copy(msg, deg_sp.at[dst_v], add=True)
            return 0
        lax.fori_loop(0, NCH, chunk, 0)

        plsc.subcore_barrier()
        pltpu.sync_copy(deg_sp.at[pl.ds(sid * ZR, ZR)],
                        degp_hbm.at[cid, pl.ds(sid * ZR, ZR)])

    return k


def _dis_tc(degp, NP):
    def body(degp_ref, dis_ref):
        dd = degp_ref[...]
        d = dd[0, :, 0] + dd[1, :, 0]
        d = d.reshape(1, -1)
        dis_ref[...] = jnp.where(d > 0, lax.rsqrt(jnp.maximum(d, 1e-12)), 0.0)
    return pl.pallas_call(
        body,
        out_shape=jax.ShapeDtypeStruct((1, NP), jnp.float32),
    )(degp)


def _mm_tc(x, w, NP):
    BM = 1024
    def body(x_ref, w_ref, o_ref):
        o_ref[...] = jnp.dot(x_ref[...], w_ref[...],
                             preferred_element_type=jnp.float32)
    return pl.pallas_call(
        body,
        grid=(NP // BM,),
        in_specs=[pl.BlockSpec((BM, D), lambda i: (i, 0)),
                  pl.BlockSpec((D, D), lambda i: (0, 0))],
        out_specs=pl.BlockSpec((BM, D), lambda i: (i, 0)),
        out_shape=jax.ShapeDtypeStruct((NP, D), jnp.float32),
    )(x, w)


def _mid_tc(pa, pb, b1, w2, NP):
    BM = 1024
    def body(pa_ref, pb_ref, b_ref, w_ref, xemb_ref, xw2_ref):
        xe = pa_ref[...] + pb_ref[...] + b_ref[...]
        xemb_ref[...] = xe
        h = jnp.maximum(xe, 0.0)
        xw2_ref[...] = jnp.dot(h, w_ref[...], preferred_element_type=jnp.float32)
    return pl.pallas_call(
        body,
        grid=(NP // BM,),
        in_specs=[pl.BlockSpec((BM, D), lambda i: (i, 0)),
                  pl.BlockSpec((BM, D), lambda i: (i, 0)),
                  pl.BlockSpec((1, D), lambda i: (0, 0)),
                  pl.BlockSpec((D, D), lambda i: (0, 0))],
        out_specs=[pl.BlockSpec((BM, D), lambda i: (i, 0)),
                   pl.BlockSpec((BM, D), lambda i: (i, 0))],
        out_shape=[jax.ShapeDtypeStruct((NP, D), jnp.float32),
                   jax.ShapeDtypeStruct((NP, D), jnp.float32)],
    )(pa, pb, b1, w2)


def _fin_tc(pa, pb, b2, NP):
    BM = 1024
    def body(pa_ref, pb_ref, b_ref, o_ref):
        o_ref[...] = pa_ref[...] + pb_ref[...] + b_ref[...]
    return pl.pallas_call(
        body,
        grid=(NP // BM,),
        in_specs=[pl.BlockSpec((BM, D), lambda i: (i, 0)),
                  pl.BlockSpec((BM, D), lambda i: (i, 0)),
                  pl.BlockSpec((1, D), lambda i: (0, 0))],
        out_specs=pl.BlockSpec((BM, D), lambda i: (i, 0)),
        out_shape=jax.ShapeDtypeStruct((NP, D), jnp.float32),
    )(pa, pb, b2)


def kernel(x, edge_index, edge_attr, W1, b1, W2, b2):
    n = x.shape[0]
    e = edge_index.shape[1]
    etot = e + n
    NP = ((n + NW * L - 1) // (NW * L)) * (NW * L)
    NCH = (etot + NW * K - 1) // (NW * K)
    EP = NCH * NW * K

    loop = jnp.arange(n, dtype=edge_index.dtype)
    src = jnp.concatenate([edge_index[0], loop])
    dst = jnp.concatenate([edge_index[1], loop])
    ew = jnp.concatenate([edge_attr, jnp.ones((n,), dtype=edge_attr.dtype)])
    src = jnp.pad(src, (0, EP - etot))
    dst = jnp.pad(dst, (0, EP - etot))
    ew = jnp.pad(ew, (0, EP - etot))
    xp = jnp.pad(x, ((0, NP - n), (0, 0)))
    b1r = b1.reshape(1, D)
    b2r = b2.reshape(1, D)

    degp = _deg_kernel(NP, NCH)(dst, ew)
    dis = _dis_tc(degp, NP)
    xw1 = _mm_tc(xp, W1, NP)
    p1 = _agg_kernel(NP, NCH)(xw1, dis, src, dst, ew)
    x_emb_p, xw2 = _mid_tc(p1[0], p1[1], b1r, W2, NP)
    p2 = _agg_kernel(NP, NCH)(xw2, dis, src, dst, ew)
    out_p = _fin_tc(p2[0], p2[1], b2r, NP)

    return (out_p[:n], x_emb_p[:n])
